# baseline (speedup 1.0000x reference)
import functools

import jax
import jax.numpy as jnp
from jax import lax
from jax.experimental import pallas as pl
from jax.experimental.pallas import tpu as pltpu
from jax.experimental.pallas import tpu_sc as plsc

LANES = 128
N_CELLS = 1024 * 1024
N_ROWS = N_CELLS // LANES
NUM_TILES = 16
PER_TILE = N_CELLS // NUM_TILES
ROWS_PER_TILE = PER_TILE // LANES

SC_BLK_ROWS = 64
SC_BLKS = ROWS_PER_TILE // SC_BLK_ROWS
EW_CHUNK = 4096
EW_CHUNKS = PER_TILE // EW_CHUNK
V16 = LANES // 16


def _build():
    mesh = plsc.VectorSubcoreMesh(
        core_axis_name="c", subcore_axis_name="s", num_cores=2, num_subcores=16
    )

    @functools.partial(
        pl.kernel,
        out_type=[
            jax.ShapeDtypeStruct((N_CELLS,), jnp.float32),
            jax.ShapeDtypeStruct((N_CELLS,), jnp.float32),
        ],
        mesh=mesh,
        scratch_types=[
            pltpu.VMEM_SHARED((N_CELLS,), jnp.float32),
            pltpu.VMEM((2, SC_BLK_ROWS, LANES), jnp.int32),
            pltpu.VMEM((2, SC_BLK_ROWS * LANES), jnp.float32),
            pltpu.VMEM((2, EW_CHUNK), jnp.float32),
            pltpu.VMEM((2, EW_CHUNK), jnp.float32),
            pltpu.VMEM((2, EW_CHUNK), jnp.float32),
            pltpu.VMEM((16,), jnp.int32),
            pltpu.SemaphoreType.DMA,
            pltpu.SemaphoreType.DMA,
            pltpu.SemaphoreType.DMA,
            pltpu.SemaphoreType.DMA,
            pltpu.SemaphoreType.DMA,
            pltpu.SemaphoreType.DMA,
        ],
    )
    def route(rflat_hbm, idx2d_hbm, it_hbm, acc_out, curw,
              acc_sh, idx_buf, cur_buf, acc_stage, ew_cur, ew_out, it_v,
              sem_idx, sem_cin, sem_sc, sem_a, sem_c, sem_o):
        cid = lax.axis_index("c")
        sid = lax.axis_index("s")
        tile_row0 = sid * ROWS_PER_TILE
        tile_base = sid * PER_TILE

        pltpu.sync_copy(it_hbm, it_v)

        @pl.when(cid == 0)
        def _init():
            pltpu.sync_copy(rflat_hbm.at[pl.ds(tile_base, PER_TILE)],
                            acc_sh.at[pl.ds(tile_base, PER_TILE)])
            pltpu.sync_copy(rflat_hbm.at[pl.ds(tile_base, PER_TILE)],
                            curw.at[pl.ds(tile_base, PER_TILE)])

        plsc.subcore_barrier()

        def one_round(_, carry):
            @pl.when(cid == 0)
            def _scatter():
                descs = [
                    pltpu.async_copy(
                        idx2d_hbm.at[pl.ds(tile_row0, SC_BLK_ROWS)],
                        idx_buf.at[0], sem_idx),
                    pltpu.async_copy(
                        curw.at[pl.ds(tile_base, SC_BLK_ROWS * LANES)],
                        cur_buf.at[0], sem_cin),
                ]
                for b in range(SC_BLKS):
                    p = b % 2
                    descs[2 * b].wait()
                    descs[2 * b + 1].wait()
                    if b + 1 < SC_BLKS:
                        descs.append(pltpu.async_copy(
                            idx2d_hbm.at[pl.ds(
                                tile_row0 + (b + 1) * SC_BLK_ROWS,
                                SC_BLK_ROWS)],
                            idx_buf.at[(b + 1) % 2], sem_idx))
                        descs.append(pltpu.async_copy(
                            curw.at[pl.ds(
                                tile_base + (b + 1) * SC_BLK_ROWS * LANES,
                                SC_BLK_ROWS * LANES)],
                            cur_buf.at[(b + 1) % 2], sem_cin))
                    sc_descs = [
                        pltpu.async_copy(
                            cur_buf.at[p, pl.ds(j * LANES, LANES)],
                            acc_sh.at[idx_buf.at[p, j]],
                            sem_sc, add=True)
                        for j in range(SC_BLK_ROWS)]
                    for dsc in sc_descs:
                        dsc.wait()

            plsc.subcore_barrier()

            @pl.when(cid == 0)
            def _elementwise():
                def ew_body(i, cc):
                    ins = []
                    for p in range(2):
                        off = tile_base + (i * 2 + p) * EW_CHUNK
                        ins.append(pltpu.async_copy(
                            acc_sh.at[pl.ds(off, EW_CHUNK)],
                            acc_stage.at[p], sem_a))
                        ins.append(pltpu.async_copy(
                            curw.at[pl.ds(off, EW_CHUNK)],
                            ew_cur.at[p], sem_c))
                    outs = []
                    for p in range(2):
                        off = tile_base + (i * 2 + p) * EW_CHUNK
                        ins[2 * p].wait()
                        ins[2 * p + 1].wait()
                        for v in range(EW_CHUNK // 16):
                            sl = pl.ds(v * 16, 16)
                            ew_out[p, sl] = acc_stage[p, sl] - ew_cur[p, sl]
                        outs.append(pltpu.async_copy(
                            ew_out.at[p], curw.at[pl.ds(off, EW_CHUNK)],
                            sem_o))
                    for o in outs:
                        o.wait()
                    return cc

                lax.fori_loop(0, EW_CHUNKS // 2, ew_body, 0)

            plsc.subcore_barrier()
            return carry

        n_rounds = it_v[pl.ds(0, 16)][0]
        lax.fori_loop(0, n_rounds, one_round, 0)

        @pl.when(cid == 0)
        def _writeout():
            pltpu.sync_copy(acc_sh.at[pl.ds(tile_base, PER_TILE)],
                            acc_out.at[pl.ds(tile_base, PER_TILE)])

    return route


_route = _build()


def kernel(runoff_generated, flow_direction_indices, iterations):
    h, w = runoff_generated.shape
    r_flat = runoff_generated.reshape(-1)
    idx_2d = flow_direction_indices.reshape(N_ROWS, LANES)
    it = jnp.full((16,), iterations, dtype=jnp.int32)
    out, _ = _route(r_flat, idx_2d, it)
    return out.reshape(h, w)


# speedup vs baseline: 1.3336x; 1.3336x over previous
import functools

import jax
import jax.numpy as jnp
from jax import lax
from jax.experimental import pallas as pl
from jax.experimental.pallas import tpu as pltpu
from jax.experimental.pallas import tpu_sc as plsc

LANES = 128
N_CELLS = 1024 * 1024
N_ROWS = N_CELLS // LANES
NUM_TILES = 16
PER_TILE = N_CELLS // NUM_TILES
ROWS_PER_TILE = PER_TILE // LANES

SC_BLK_ROWS = 64
SC_BLKS = ROWS_PER_TILE // SC_BLK_ROWS
EW_CHUNK = 4096
EW_CHUNKS = PER_TILE // EW_CHUNK
V16 = LANES // 16


def _build():
    mesh = plsc.VectorSubcoreMesh(
        core_axis_name="c", subcore_axis_name="s", num_cores=2, num_subcores=16
    )

    @functools.partial(
        pl.kernel,
        out_type=[
            jax.ShapeDtypeStruct((N_CELLS,), jnp.float32),
            jax.ShapeDtypeStruct((N_CELLS,), jnp.float32),
        ],
        mesh=mesh,
        scratch_types=[
            pltpu.VMEM_SHARED((N_CELLS,), jnp.float32),
            pltpu.VMEM((2, SC_BLK_ROWS, LANES), jnp.int32),
            pltpu.VMEM((2, SC_BLK_ROWS * LANES), jnp.float32),
            pltpu.VMEM((2, EW_CHUNK), jnp.float32),
            pltpu.VMEM((2, EW_CHUNK), jnp.float32),
            pltpu.VMEM((2, EW_CHUNK), jnp.float32),
            pltpu.VMEM((16,), jnp.int32),
            pltpu.SemaphoreType.DMA,
            pltpu.SemaphoreType.DMA,
            pltpu.SemaphoreType.DMA,
            pltpu.SemaphoreType.DMA,
            pltpu.SemaphoreType.DMA,
            pltpu.SemaphoreType.DMA,
        ],
    )
    def route(rflat_hbm, idx2d_hbm, it_hbm, acc_out, curw,
              acc_sh, idx_buf, cur_buf, acc_stage, ew_cur, ew_out, it_v,
              sem_idx, sem_cin, sem_sc, sem_a, sem_c, sem_o):
        cid = lax.axis_index("c")
        sid = lax.axis_index("s")
        tile_row0 = sid * ROWS_PER_TILE
        tile_base = sid * PER_TILE

        pltpu.sync_copy(it_hbm, it_v)

        @pl.when(cid == 0)
        def _init():
            pltpu.sync_copy(rflat_hbm.at[pl.ds(tile_base, PER_TILE)],
                            acc_sh.at[pl.ds(tile_base, PER_TILE)])
            pltpu.sync_copy(rflat_hbm.at[pl.ds(tile_base, PER_TILE)],
                            curw.at[pl.ds(tile_base, PER_TILE)])

        plsc.subcore_barrier()

        def one_round(_, carry):
            @pl.when(cid == 0)
            def _scatter():
                descs = [
                    pltpu.async_copy(
                        idx2d_hbm.at[pl.ds(tile_row0, SC_BLK_ROWS)],
                        idx_buf.at[0], sem_idx),
                    pltpu.async_copy(
                        curw.at[pl.ds(tile_base, SC_BLK_ROWS * LANES)],
                        cur_buf.at[0], sem_cin),
                ]
                for b in range(SC_BLKS):
                    p = b % 2
                    descs[2 * b].wait()
                    descs[2 * b + 1].wait()
                    if b + 1 < SC_BLKS:
                        descs.append(pltpu.async_copy(
                            idx2d_hbm.at[pl.ds(
                                tile_row0 + (b + 1) * SC_BLK_ROWS,
                                SC_BLK_ROWS)],
                            idx_buf.at[(b + 1) % 2], sem_idx))
                        descs.append(pltpu.async_copy(
                            curw.at[pl.ds(
                                tile_base + (b + 1) * SC_BLK_ROWS * LANES,
                                SC_BLK_ROWS * LANES)],
                            cur_buf.at[(b + 1) % 2], sem_cin))
                    sc_descs = [
                        pltpu.async_copy(
                            cur_buf.at[p, pl.ds(j * LANES, LANES)],
                            acc_sh.at[idx_buf.at[p, j]],
                            sem_sc, add=True)
                        for j in range(SC_BLK_ROWS)]
                    for dsc in sc_descs:
                        dsc.wait()

            plsc.subcore_barrier()

            @pl.when(cid == 0)
            def _elementwise():
                def ew_body(i, cc):
                    ins = []
                    for p in range(2):
                        off = tile_base + (i * 2 + p) * EW_CHUNK
                        ins.append(pltpu.async_copy(
                            acc_sh.at[pl.ds(off, EW_CHUNK)],
                            acc_stage.at[p], sem_a))
                        ins.append(pltpu.async_copy(
                            curw.at[pl.ds(off, EW_CHUNK)],
                            ew_cur.at[p], sem_c))
                    outs = []
                    for p in range(2):
                        off = tile_base + (i * 2 + p) * EW_CHUNK
                        ins[2 * p].wait()
                        ins[2 * p + 1].wait()
                        @plsc.parallel_loop(0, EW_CHUNK, 16, unroll=8)
                        def _ew(v):
                            sl = pl.ds(v, 16)
                            ew_out[p, sl] = acc_stage[p, sl] - ew_cur[p, sl]
                        outs.append(pltpu.async_copy(
                            ew_out.at[p], curw.at[pl.ds(off, EW_CHUNK)],
                            sem_o))
                    for o in outs:
                        o.wait()
                    return cc

                lax.fori_loop(0, EW_CHUNKS // 2, ew_body, 0)

            plsc.subcore_barrier()
            return carry

        n_rounds = it_v[pl.ds(0, 16)][0]
        lax.fori_loop(0, n_rounds, one_round, 0)

        @pl.when(cid == 0)
        def _writeout():
            pltpu.sync_copy(acc_sh.at[pl.ds(tile_base, PER_TILE)],
                            acc_out.at[pl.ds(tile_base, PER_TILE)])

    return route


_route = _build()


def kernel(runoff_generated, flow_direction_indices, iterations):
    h, w = runoff_generated.shape
    r_flat = runoff_generated.reshape(-1)
    idx_2d = flow_direction_indices.reshape(N_ROWS, LANES)
    it = jnp.full((16,), iterations, dtype=jnp.int32)
    out, _ = _route(r_flat, idx_2d, it)
    return out.reshape(h, w)


# speedup vs baseline: 1.4846x; 1.1133x over previous
import functools

import jax
import jax.numpy as jnp
from jax import lax
from jax.experimental import pallas as pl
from jax.experimental.pallas import tpu as pltpu
from jax.experimental.pallas import tpu_sc as plsc

LANES = 128
N_CELLS = 1024 * 1024
N_ROWS = N_CELLS // LANES
NUM_TILES = 16
PER_TILE = N_CELLS // NUM_TILES
ROWS_PER_TILE = PER_TILE // LANES

NRING = 4
SC_BLK_ROWS = 32
SC_BLKS = ROWS_PER_TILE // SC_BLK_ROWS
SC_BLK = SC_BLK_ROWS * LANES
EW_CHUNK = 2048
EW_CHUNKS = PER_TILE // EW_CHUNK


def _build():
    mesh = plsc.VectorSubcoreMesh(
        core_axis_name="c", subcore_axis_name="s", num_cores=2, num_subcores=16
    )

    @functools.partial(
        pl.kernel,
        out_type=[
            jax.ShapeDtypeStruct((N_CELLS,), jnp.float32),
            jax.ShapeDtypeStruct((N_CELLS,), jnp.float32),
        ],
        mesh=mesh,
        scratch_types=[
            pltpu.VMEM_SHARED((N_CELLS,), jnp.float32),
            pltpu.VMEM((NRING, SC_BLK_ROWS, LANES), jnp.int32),
            pltpu.VMEM((NRING, SC_BLK), jnp.float32),
            pltpu.VMEM((NRING, EW_CHUNK), jnp.float32),
            pltpu.VMEM((NRING, EW_CHUNK), jnp.float32),
            pltpu.VMEM((NRING, EW_CHUNK), jnp.float32),
            pltpu.VMEM((16,), jnp.int32),
            pltpu.SemaphoreType.DMA,
            pltpu.SemaphoreType.DMA,
            pltpu.SemaphoreType.DMA,
            pltpu.SemaphoreType.DMA,
            pltpu.SemaphoreType.DMA,
            pltpu.SemaphoreType.DMA,
        ],
    )
    def route(rflat_hbm, idx2d_hbm, it_hbm, acc_out, curw,
              acc_sh, idx_buf, cur_buf, acc_stage, ew_cur, ew_out, it_v,
              sem_idx, sem_cin, sem_sc, sem_a, sem_c, sem_o):
        cid = lax.axis_index("c")
        sid = lax.axis_index("s")
        tile_row0 = sid * ROWS_PER_TILE
        tile_base = sid * PER_TILE

        pltpu.sync_copy(it_hbm, it_v)

        @pl.when(cid == 0)
        def _init():
            pltpu.sync_copy(rflat_hbm.at[pl.ds(tile_base, PER_TILE)],
                            acc_sh.at[pl.ds(tile_base, PER_TILE)])
            pltpu.sync_copy(rflat_hbm.at[pl.ds(tile_base, PER_TILE)],
                            curw.at[pl.ds(tile_base, PER_TILE)])

        plsc.subcore_barrier()

        def _issue_sc_in(b):
            return (
                pltpu.async_copy(
                    idx2d_hbm.at[pl.ds(tile_row0 + b * SC_BLK_ROWS,
                                       SC_BLK_ROWS)],
                    idx_buf.at[b % NRING], sem_idx),
                pltpu.async_copy(
                    curw.at[pl.ds(tile_base + b * SC_BLK, SC_BLK)],
                    cur_buf.at[b % NRING], sem_cin),
            )

        def _issue_ew_in(c):
            off = tile_base + c * EW_CHUNK
            return (
                pltpu.async_copy(acc_sh.at[pl.ds(off, EW_CHUNK)],
                                 acc_stage.at[c % NRING], sem_a),
                pltpu.async_copy(curw.at[pl.ds(off, EW_CHUNK)],
                                 ew_cur.at[c % NRING], sem_c),
            )

        def one_round(_, carry):
            @pl.when(cid == 0)
            def _scatter():
                def _drain_sc(p):
                    pltpu.make_async_copy(
                        rflat_hbm.at[pl.ds(0, SC_BLK)],
                        cur_buf.at[p], sem_sc).wait()

                ins = [_issue_sc_in(b) for b in range(NRING - 1)]
                for b in range(SC_BLKS):
                    nxt = b + NRING - 1
                    if nxt < SC_BLKS:
                        if nxt - NRING >= 0:
                            _drain_sc(nxt % NRING)
                        ins.append(_issue_sc_in(nxt))
                    ins[b][0].wait()
                    ins[b][1].wait()
                    p = b % NRING
                    for j in range(SC_BLK_ROWS):
                        pltpu.async_copy(
                            cur_buf.at[p, pl.ds(j * LANES, LANES)],
                            acc_sh.at[idx_buf.at[p, j]],
                            sem_sc, add=True)
                for b in range(max(0, SC_BLKS - NRING), SC_BLKS):
                    _drain_sc(b % NRING)

            plsc.subcore_barrier()

            @pl.when(cid == 0)
            def _elementwise():
                ins = [_issue_ew_in(c) for c in range(NRING - 1)]
                o_d = [None] * EW_CHUNKS
                for c in range(EW_CHUNKS):
                    nxt = c + NRING - 1
                    if nxt < EW_CHUNKS:
                        ins.append(_issue_ew_in(nxt))
                    ins[c][0].wait()
                    ins[c][1].wait()
                    if c - NRING >= 0:
                        o_d[c - NRING].wait()
                    p = c % NRING

                    @plsc.parallel_loop(0, EW_CHUNK, 16, unroll=8)
                    def _ew(v):
                        sl = pl.ds(v, 16)
                        ew_out[p, sl] = acc_stage[p, sl] - ew_cur[p, sl]

                    o_d[c] = pltpu.async_copy(
                        ew_out.at[p],
                        curw.at[pl.ds(tile_base + c * EW_CHUNK, EW_CHUNK)],
                        sem_o)
                for c in range(max(0, EW_CHUNKS - NRING), EW_CHUNKS):
                    o_d[c].wait()

            plsc.subcore_barrier()
            return carry

        n_rounds = it_v[pl.ds(0, 16)][0]
        lax.fori_loop(0, n_rounds, one_round, 0)

        @pl.when(cid == 0)
        def _writeout():
            pltpu.sync_copy(acc_sh.at[pl.ds(tile_base, PER_TILE)],
                            acc_out.at[pl.ds(tile_base, PER_TILE)])

    return route


_route = _build()


def kernel(runoff_generated, flow_direction_indices, iterations):
    h, w = runoff_generated.shape
    r_flat = runoff_generated.reshape(-1)
    idx_2d = flow_direction_indices.reshape(N_ROWS, LANES)
    it = jnp.full((16,), iterations, dtype=jnp.int32)
    out, _ = _route(r_flat, idx_2d, it)
    return out.reshape(h, w)


# speedup vs baseline: 2.2051x; 1.4853x over previous
import functools

import jax
import jax.numpy as jnp
from jax import lax
from jax.experimental import pallas as pl
from jax.experimental.pallas import tpu as pltpu
from jax.experimental.pallas import tpu_sc as plsc

LANES = 128
N_CELLS = 1024 * 1024
N_ROWS = N_CELLS // LANES
NUM_TILES = 16
PER_TILE = N_CELLS // NUM_TILES
ROWS_PER_TILE = PER_TILE // LANES

NRING = 4
SC_BLK_ROWS = 32
SC_BLKS = ROWS_PER_TILE // SC_BLK_ROWS
SC_BLK = SC_BLK_ROWS * LANES
EW_CHUNK = 2048
EW_CHUNKS = PER_TILE // EW_CHUNK

POLY_ITERS = 32


def _poly_coeffs(n):
    a, c = [1], [1]
    for _ in range(n):
        an = [0] * (len(c) + 1)
        cn = [0] * (len(c) + 1)
        for i, v in enumerate(a):
            an[i] += v
            cn[i] += v
        for i, v in enumerate(c):
            an[i + 1] += v
            cn[i + 1] += v
            cn[i] -= v
        a, c = an, cn
    return a


_A32 = _poly_coeffs(POLY_ITERS)
_EVEN = _A32[0::2]
_ODD = _A32[1::2] + [0]
_HSTEPS = len(_EVEN) - 1


mesh = plsc.VectorSubcoreMesh(
    core_axis_name="c", subcore_axis_name="s", num_cores=2, num_subcores=16
)


@functools.partial(
    pl.kernel,
    out_type=[
        jax.ShapeDtypeStruct((2 * N_CELLS,), jnp.float32),
        jax.ShapeDtypeStruct((2 * N_ROWS, LANES), jnp.int32),
    ],
    mesh=mesh,
    scratch_types=[
        pltpu.VMEM_SHARED((N_CELLS,), jnp.float32),
        pltpu.VMEM((2 * SC_BLK,), jnp.int32),
        pltpu.VMEM((2, SC_BLK_ROWS, LANES), jnp.int32),
        pltpu.VMEM((NRING, SC_BLK_ROWS, LANES), jnp.int32),
        pltpu.VMEM((NRING, SC_BLK), jnp.float32),
        pltpu.VMEM((3 * EW_CHUNK,), jnp.float32),
        pltpu.VMEM((3 * EW_CHUNK,), jnp.float32),
        pltpu.VMEM((32, LANES), jnp.float32),
        pltpu.SemaphoreType.DMA,
        pltpu.SemaphoreType.DMA,
        pltpu.SemaphoreType.DMA,
        pltpu.SemaphoreType.DMA,
        pltpu.SemaphoreType.DMA,
        pltpu.SemaphoreType.DMA,
    ],
)
def _horner(rflat_hbm, idxflat_hbm, tbl_hbm, w_hbm, idx2_hbm,
            t_sh, ibuf, gdst, idx_ring, wval, stage, rbuf, tbl,
            sem_in, sem_w, sem_sc, sem_so, sem_r, sem_o):
    cid = lax.axis_index("c")
    sid = lax.axis_index("s")
    tile_row0 = sid * ROWS_PER_TILE
    tile_base = sid * PER_TILE
    w_base = cid * N_CELLS + tile_base
    i2_row0 = cid * N_ROWS + tile_row0

    pltpu.sync_copy(tbl_hbm.at[cid], tbl)

    c_top = tbl[16, pl.ds(0, 16)]
    c_next = tbl[15, pl.ds(0, 16)]

    pro = [pltpu.async_copy(
        idxflat_hbm.at[pl.ds(tile_row0 * LANES, SC_BLK)],
        ibuf.at[pl.ds(0, SC_BLK)], sem_in)]
    g_d = [None] * SC_BLKS
    o_d = [None] * SC_BLKS
    for b in range(SC_BLKS):
        p = b % 2
        if b + 1 < SC_BLKS:
            pro.append(pltpu.async_copy(
                idxflat_hbm.at[pl.ds(
                    (tile_row0 + (b + 1) * SC_BLK_ROWS) * LANES, SC_BLK)],
                ibuf.at[pl.ds(((b + 1) % 2) * SC_BLK, SC_BLK)], sem_in))
        pro[b].wait()
        if b - 2 >= 0:
            o_d[b - 2].wait()
        g_d[b] = [pltpu.async_copy(
            idxflat_hbm.at[ibuf.at[pl.ds(p * SC_BLK + j * LANES, LANES)]],
            gdst.at[p, j], sem_sc)
            for j in range(SC_BLK_ROWS)]
        for dsc in g_d[b]:
            dsc.wait()
        o_d[b] = pltpu.async_copy(
            gdst.at[p],
            idx2_hbm.at[pl.ds(i2_row0 + b * SC_BLK_ROWS, SC_BLK_ROWS)],
            sem_o)
    o_d[SC_BLKS - 2].wait()
    o_d[SC_BLKS - 1].wait()

    rin = [pltpu.async_copy(
        rflat_hbm.at[pl.ds(tile_base, EW_CHUNK)],
        rbuf.at[pl.ds(0, EW_CHUNK)], sem_r)]
    wo_d = [None] * EW_CHUNKS
    to_d = [None] * EW_CHUNKS
    for c in range(EW_CHUNKS):
        if c + 1 < EW_CHUNKS:
            if c - 1 >= 0:
                wo_d[c - 1].wait()
                to_d[c - 1].wait()
            rin.append(pltpu.async_copy(
                rflat_hbm.at[pl.ds(tile_base + (c + 1) * EW_CHUNK, EW_CHUNK)],
                rbuf.at[pl.ds(((c + 1) % 3) * EW_CHUNK, EW_CHUNK)], sem_r))
        rin[c].wait()
        so = (c % 3) * EW_CHUNK

        @plsc.parallel_loop(0, EW_CHUNK, 16, unroll=8)
        def _sc_top(v):
            stage[pl.ds(so + v, 16)] = rbuf[pl.ds(so + v, 16)] * c_top

        wo_d[c] = pltpu.async_copy(
            stage.at[pl.ds(so, EW_CHUNK)],
            w_hbm.at[pl.ds(w_base + c * EW_CHUNK, EW_CHUNK)],
            sem_so)

        @plsc.parallel_loop(0, EW_CHUNK, 16, unroll=8)
        def _sc_nxt(v):
            rbuf[pl.ds(so + v, 16)] = rbuf[pl.ds(so + v, 16)] * c_next

        to_d[c] = pltpu.async_copy(
            rbuf.at[pl.ds(so, EW_CHUNK)],
            t_sh.at[pl.ds(tile_base + c * EW_CHUNK, EW_CHUNK)],
            sem_o)
    for c in range(EW_CHUNKS - 2, EW_CHUNKS):
        wo_d[c].wait()
        to_d[c].wait()

    plsc.subcore_barrier()

    idx2_2d = idx2_hbm

    def one_round(i, carry):
        m = (_HSTEPS - 1) - i
        mm = lax.max(m - 1, 0)
        coef = tbl[mm, pl.ds(0, 16)]

        def _issue_in(b):
            return (
                pltpu.async_copy(
                    idx2_2d.at[pl.ds(i2_row0 + b * SC_BLK_ROWS, SC_BLK_ROWS)],
                    idx_ring.at[b % NRING], sem_in),
                pltpu.async_copy(
                    w_hbm.at[pl.ds(w_base + b * SC_BLK, SC_BLK)],
                    wval.at[b % NRING], sem_w),
            )

        def _drain_sc(p):
            pltpu.make_async_copy(
                rflat_hbm.at[pl.ds(0, SC_BLK)], wval.at[p], sem_sc).wait()

        ins = [_issue_in(b) for b in range(NRING - 1)]
        for b in range(SC_BLKS):
            nxt = b + NRING - 1
            if nxt < SC_BLKS:
                if nxt - NRING >= 0:
                    _drain_sc(nxt % NRING)
                ins.append(_issue_in(nxt))
            ins[b][0].wait()
            ins[b][1].wait()
            p = b % NRING
            for j in range(SC_BLK_ROWS):
                pltpu.async_copy(
                    wval.at[p, pl.ds(j * LANES, LANES)],
                    t_sh.at[idx_ring.at[p, j]],
                    sem_sc, add=True)
        for b in range(max(0, SC_BLKS - NRING), SC_BLKS):
            _drain_sc(b % NRING)

        plsc.subcore_barrier()

        rin2 = [pltpu.async_copy(
            rflat_hbm.at[pl.ds(tile_base, EW_CHUNK)],
            rbuf.at[pl.ds(0, EW_CHUNK)], sem_r)]
        st_in = [pltpu.async_copy(
            t_sh.at[pl.ds(tile_base, EW_CHUNK)],
            stage.at[pl.ds(0, EW_CHUNK)], sem_in)]
        for c in range(EW_CHUNKS):
            so = (c % 3) * EW_CHUNK
            if c + 1 < EW_CHUNKS:
                no = ((c + 1) % 3) * EW_CHUNK
                rin2.append(pltpu.async_copy(
                    rflat_hbm.at[pl.ds(tile_base + (c + 1) * EW_CHUNK,
                                       EW_CHUNK)],
                    rbuf.at[pl.ds(no, EW_CHUNK)], sem_r))
                st_in.append(pltpu.async_copy(
                    t_sh.at[pl.ds(tile_base + (c + 1) * EW_CHUNK, EW_CHUNK)],
                    stage.at[pl.ds(no, EW_CHUNK)], sem_in))
            st_in[c].wait()
            pltpu.sync_copy(
                stage.at[pl.ds(so, EW_CHUNK)],
                w_hbm.at[pl.ds(w_base + c * EW_CHUNK, EW_CHUNK)])
            rin2[c].wait()

            @plsc.parallel_loop(0, EW_CHUNK, 16, unroll=8)
            def _scale(v):
                rbuf[pl.ds(so + v, 16)] = rbuf[pl.ds(so + v, 16)] * coef

            pltpu.sync_copy(
                rbuf.at[pl.ds(so, EW_CHUNK)],
                t_sh.at[pl.ds(tile_base + c * EW_CHUNK, EW_CHUNK)])

        plsc.subcore_barrier()
        return carry

    lax.fori_loop(0, _HSTEPS, one_round, 0)


@functools.partial(
    pl.kernel,
    out_type=jax.ShapeDtypeStruct((N_CELLS,), jnp.float32),
    mesh=mesh,
    scratch_types=[
        pltpu.VMEM_SHARED((N_CELLS,), jnp.float32),
        pltpu.VMEM((NRING, SC_BLK_ROWS, LANES), jnp.int32),
        pltpu.VMEM((NRING, SC_BLK), jnp.float32),
        pltpu.VMEM((3, EW_CHUNK), jnp.float32),
        pltpu.SemaphoreType.DMA,
        pltpu.SemaphoreType.DMA,
        pltpu.SemaphoreType.DMA,
        pltpu.SemaphoreType.DMA,
    ],
)
def _combine(w_hbm, idx2d_hbm, acc_out,
             acc_sh, idx_ring, uval, stage, sem_in, sem_w, sem_sc, sem_o):
    cid = lax.axis_index("c")
    sid = lax.axis_index("s")
    tile_row0 = sid * ROWS_PER_TILE
    tile_base = sid * PER_TILE

    @pl.when(cid == 0)
    def _init():
        pltpu.sync_copy(w_hbm.at[pl.ds(tile_base, PER_TILE)],
                        acc_sh.at[pl.ds(tile_base, PER_TILE)])

    plsc.subcore_barrier()

    @pl.when(cid == 0)
    def _scatter():
        def _issue_in(b):
            return (
                pltpu.async_copy(
                    idx2d_hbm.at[pl.ds(tile_row0 + b * SC_BLK_ROWS,
                                       SC_BLK_ROWS)],
                    idx_ring.at[b % NRING], sem_in),
                pltpu.async_copy(
                    w_hbm.at[pl.ds(N_CELLS + tile_base + b * SC_BLK, SC_BLK)],
                    uval.at[b % NRING], sem_w),
            )

        def _drain_sc(p):
            pltpu.make_async_copy(
                w_hbm.at[pl.ds(0, SC_BLK)], uval.at[p], sem_sc).wait()

        ins = [_issue_in(b) for b in range(NRING - 1)]
        for b in range(SC_BLKS):
            nxt = b + NRING - 1
            if nxt < SC_BLKS:
                if nxt - NRING >= 0:
                    _drain_sc(nxt % NRING)
                ins.append(_issue_in(nxt))
            ins[b][0].wait()
            ins[b][1].wait()
            p = b % NRING
            for j in range(SC_BLK_ROWS):
                pltpu.async_copy(
                    uval.at[p, pl.ds(j * LANES, LANES)],
                    acc_sh.at[idx_ring.at[p, j]],
                    sem_sc, add=True)
        for b in range(max(0, SC_BLKS - NRING), SC_BLKS):
            _drain_sc(b % NRING)

    plsc.subcore_barrier()

    @pl.when(cid == 0)
    def _writeout():
        pltpu.sync_copy(acc_sh.at[pl.ds(tile_base, PER_TILE)],
                        acc_out.at[pl.ds(tile_base, PER_TILE)])


@functools.partial(
    pl.kernel,
    out_type=[
        jax.ShapeDtypeStruct((N_CELLS,), jnp.float32),
        jax.ShapeDtypeStruct((N_CELLS,), jnp.float32),
    ],
    mesh=mesh,
    scratch_types=[
        pltpu.VMEM_SHARED((N_CELLS,), jnp.float32),
        pltpu.VMEM((NRING, SC_BLK_ROWS, LANES), jnp.int32),
        pltpu.VMEM((NRING, SC_BLK), jnp.float32),
        pltpu.VMEM((NRING, EW_CHUNK), jnp.float32),
        pltpu.VMEM((NRING, EW_CHUNK), jnp.float32),
        pltpu.VMEM((NRING, EW_CHUNK), jnp.float32),
        pltpu.VMEM((16,), jnp.int32),
        pltpu.SemaphoreType.DMA,
        pltpu.SemaphoreType.DMA,
        pltpu.SemaphoreType.DMA,
        pltpu.SemaphoreType.DMA,
        pltpu.SemaphoreType.DMA,
        pltpu.SemaphoreType.DMA,
    ],
)
def _route(rflat_hbm, idx2d_hbm, it_hbm, acc_out, curw,
           acc_sh, idx_buf, cur_buf, acc_stage, ew_cur, ew_out, it_v,
           sem_idx, sem_cin, sem_sc, sem_a, sem_c, sem_o):
    cid = lax.axis_index("c")
    sid = lax.axis_index("s")
    tile_row0 = sid * ROWS_PER_TILE
    tile_base = sid * PER_TILE

    pltpu.sync_copy(it_hbm, it_v)

    @pl.when(cid == 0)
    def _init():
        pltpu.sync_copy(rflat_hbm.at[pl.ds(tile_base, PER_TILE)],
                        acc_sh.at[pl.ds(tile_base, PER_TILE)])
        pltpu.sync_copy(rflat_hbm.at[pl.ds(tile_base, PER_TILE)],
                        curw.at[pl.ds(tile_base, PER_TILE)])

    plsc.subcore_barrier()

    def _issue_sc_in(b):
        return (
            pltpu.async_copy(
                idx2d_hbm.at[pl.ds(tile_row0 + b * SC_BLK_ROWS, SC_BLK_ROWS)],
                idx_buf.at[b % NRING], sem_idx),
            pltpu.async_copy(
                curw.at[pl.ds(tile_base + b * SC_BLK, SC_BLK)],
                cur_buf.at[b % NRING], sem_cin),
        )

    def _issue_ew_in(c):
        off = tile_base + c * EW_CHUNK
        return (
            pltpu.async_copy(acc_sh.at[pl.ds(off, EW_CHUNK)],
                             acc_stage.at[c % NRING], sem_a),
            pltpu.async_copy(curw.at[pl.ds(off, EW_CHUNK)],
                             ew_cur.at[c % NRING], sem_c),
        )

    def one_round(_, carry):
        @pl.when(cid == 0)
        def _scatter():
            def _drain_sc(p):
                pltpu.make_async_copy(
                    rflat_hbm.at[pl.ds(0, SC_BLK)],
                    cur_buf.at[p], sem_sc).wait()

            ins = [_issue_sc_in(b) for b in range(NRING - 1)]
            for b in range(SC_BLKS):
                nxt = b + NRING - 1
                if nxt < SC_BLKS:
                    if nxt - NRING >= 0:
                        _drain_sc(nxt % NRING)
                    ins.append(_issue_sc_in(nxt))
                ins[b][0].wait()
                ins[b][1].wait()
                p = b % NRING
                for j in range(SC_BLK_ROWS):
                    pltpu.async_copy(
                        cur_buf.at[p, pl.ds(j * LANES, LANES)],
                        acc_sh.at[idx_buf.at[p, j]],
                        sem_sc, add=True)
            for b in range(max(0, SC_BLKS - NRING), SC_BLKS):
                _drain_sc(b % NRING)

        plsc.subcore_barrier()

        @pl.when(cid == 0)
        def _elementwise():
            ins = [_issue_ew_in(c) for c in range(NRING - 1)]
            o_d = [None] * EW_CHUNKS
            for c in range(EW_CHUNKS):
                nxt = c + NRING - 1
                if nxt < EW_CHUNKS:
                    ins.append(_issue_ew_in(nxt))
                ins[c][0].wait()
                ins[c][1].wait()
                if c - NRING >= 0:
                    o_d[c - NRING].wait()
                p = c % NRING

                @plsc.parallel_loop(0, EW_CHUNK, 16, unroll=8)
                def _ew(v):
                    sl = pl.ds(v, 16)
                    ew_out[p, sl] = acc_stage[p, sl] - ew_cur[p, sl]

                o_d[c] = pltpu.async_copy(
                    ew_out.at[p],
                    curw.at[pl.ds(tile_base + c * EW_CHUNK, EW_CHUNK)],
                    sem_o)
            for c in range(max(0, EW_CHUNKS - NRING), EW_CHUNKS):
                o_d[c].wait()

        plsc.subcore_barrier()
        return carry

    n_rounds = it_v[pl.ds(0, 16)][0]
    lax.fori_loop(0, n_rounds, one_round, 0)

    @pl.when(cid == 0)
    def _writeout():
        pltpu.sync_copy(acc_sh.at[pl.ds(tile_base, PER_TILE)],
                        acc_out.at[pl.ds(tile_base, PER_TILE)])


def _tbl_host():
    t = jnp.zeros((2, 32, LANES), jnp.float32)
    e = (jnp.array(_EVEN, jnp.float32)[:, None]
         * jnp.ones((1, LANES), jnp.float32))
    o = (jnp.array(_ODD, jnp.float32)[:, None]
         * jnp.ones((1, LANES), jnp.float32))
    t = t.at[0, : len(_EVEN)].set(e)
    t = t.at[1, : len(_ODD)].set(o)
    return t


def kernel(runoff_generated, flow_direction_indices, iterations):
    h, w = runoff_generated.shape
    r_flat = runoff_generated.reshape(-1)
    idx_flat = flow_direction_indices.reshape(-1)
    idx_2d = flow_direction_indices.reshape(N_ROWS, LANES)

    def fast(_):
        tbl = _tbl_host()
        w_buf, _i2 = _horner(r_flat, idx_flat, tbl)
        return _combine(w_buf, idx_2d)

    def general(_):
        it = jnp.full((16,), iterations, dtype=jnp.int32)
        out, _cw = _route(r_flat, idx_2d, it)
        return out

    it_scalar = jnp.asarray(iterations, jnp.int32)
    out = lax.cond(it_scalar == POLY_ITERS, fast, general, 0)
    return out.reshape(h, w)


# speedup vs baseline: 2.2110x; 1.0027x over previous
import functools

import jax
import jax.numpy as jnp
from jax import lax
from jax.experimental import pallas as pl
from jax.experimental.pallas import tpu as pltpu
from jax.experimental.pallas import tpu_sc as plsc

LANES = 128
N_CELLS = 1024 * 1024
N_ROWS = N_CELLS // LANES
NUM_TILES = 16
PER_TILE = N_CELLS // NUM_TILES
ROWS_PER_TILE = PER_TILE // LANES

NRING = 4
SC_BLK_ROWS = 32
SC_BLKS = ROWS_PER_TILE // SC_BLK_ROWS
SC_BLK = SC_BLK_ROWS * LANES
EW_CHUNK = 2048
EW_CHUNKS = PER_TILE // EW_CHUNK

POLY_ITERS = 32


def _poly_coeffs(n):
    a, c = [1], [1]
    for _ in range(n):
        an = [0] * (len(c) + 1)
        cn = [0] * (len(c) + 1)
        for i, v in enumerate(a):
            an[i] += v
            cn[i] += v
        for i, v in enumerate(c):
            an[i + 1] += v
            cn[i + 1] += v
            cn[i] -= v
        a, c = an, cn
    return a


_A32 = _poly_coeffs(POLY_ITERS)
_EVEN = _A32[0::2]
_ODD = _A32[1::2] + [0]
_HSTEPS = len(_EVEN) - 1


mesh = plsc.VectorSubcoreMesh(
    core_axis_name="c", subcore_axis_name="s", num_cores=2, num_subcores=16
)


@functools.partial(
    pl.kernel,
    out_type=[
        jax.ShapeDtypeStruct((2 * N_CELLS,), jnp.float32),
        jax.ShapeDtypeStruct((2 * N_ROWS, LANES), jnp.int32),
    ],
    mesh=mesh,
    scratch_types=[
        pltpu.VMEM_SHARED((N_CELLS,), jnp.float32),
        pltpu.VMEM((2 * SC_BLK,), jnp.int32),
        pltpu.VMEM((2, SC_BLK_ROWS, LANES), jnp.int32),
        pltpu.VMEM((NRING, SC_BLK_ROWS, LANES), jnp.int32),
        pltpu.VMEM((NRING, SC_BLK), jnp.float32),
        pltpu.VMEM((3 * EW_CHUNK,), jnp.float32),
        pltpu.VMEM((3 * EW_CHUNK,), jnp.float32),
        pltpu.VMEM((32, LANES), jnp.float32),
        pltpu.SemaphoreType.DMA,
        pltpu.SemaphoreType.DMA,
        pltpu.SemaphoreType.DMA,
        pltpu.SemaphoreType.DMA,
        pltpu.SemaphoreType.DMA,
        pltpu.SemaphoreType.DMA,
    ],
)
def _horner(rflat_hbm, idxflat_hbm, tbl_hbm, w_hbm, idx2_hbm,
            t_sh, ibuf, gdst, idx_ring, wval, stage, rbuf, tbl,
            sem_in, sem_w, sem_sc, sem_so, sem_r, sem_o):
    cid = lax.axis_index("c")
    sid = lax.axis_index("s")
    tile_row0 = sid * ROWS_PER_TILE
    tile_base = sid * PER_TILE
    w_base = cid * N_CELLS + tile_base
    i2_row0 = cid * N_ROWS + tile_row0

    pltpu.sync_copy(tbl_hbm.at[cid], tbl)

    c_top = tbl[16, pl.ds(0, 16)]
    c_next = tbl[15, pl.ds(0, 16)]

    pro = [pltpu.async_copy(
        idxflat_hbm.at[pl.ds(tile_row0 * LANES, SC_BLK)],
        ibuf.at[pl.ds(0, SC_BLK)], sem_in)]
    g_d = [None] * SC_BLKS
    o_d = [None] * SC_BLKS
    for b in range(SC_BLKS):
        p = b % 2
        if b + 1 < SC_BLKS:
            pro.append(pltpu.async_copy(
                idxflat_hbm.at[pl.ds(
                    (tile_row0 + (b + 1) * SC_BLK_ROWS) * LANES, SC_BLK)],
                ibuf.at[pl.ds(((b + 1) % 2) * SC_BLK, SC_BLK)], sem_in))
        pro[b].wait()
        if b - 2 >= 0:
            o_d[b - 2].wait()
        g_d[b] = [pltpu.async_copy(
            idxflat_hbm.at[ibuf.at[pl.ds(p * SC_BLK + j * LANES, LANES)]],
            gdst.at[p, j], sem_sc)
            for j in range(SC_BLK_ROWS)]
        for dsc in g_d[b]:
            dsc.wait()
        o_d[b] = pltpu.async_copy(
            gdst.at[p],
            idx2_hbm.at[pl.ds(i2_row0 + b * SC_BLK_ROWS, SC_BLK_ROWS)],
            sem_o)
    o_d[SC_BLKS - 2].wait()
    o_d[SC_BLKS - 1].wait()

    rin = [pltpu.async_copy(
        rflat_hbm.at[pl.ds(tile_base, EW_CHUNK)],
        rbuf.at[pl.ds(0, EW_CHUNK)], sem_r)]
    wo_d = [None] * EW_CHUNKS
    to_d = [None] * EW_CHUNKS
    for c in range(EW_CHUNKS):
        if c + 1 < EW_CHUNKS:
            if c - 1 >= 0:
                wo_d[c - 1].wait()
                to_d[c - 1].wait()
            rin.append(pltpu.async_copy(
                rflat_hbm.at[pl.ds(tile_base + (c + 1) * EW_CHUNK, EW_CHUNK)],
                rbuf.at[pl.ds(((c + 1) % 3) * EW_CHUNK, EW_CHUNK)], sem_r))
        rin[c].wait()
        so = (c % 3) * EW_CHUNK

        @plsc.parallel_loop(0, EW_CHUNK, 16, unroll=8)
        def _sc_top(v):
            stage[pl.ds(so + v, 16)] = rbuf[pl.ds(so + v, 16)] * c_top

        wo_d[c] = pltpu.async_copy(
            stage.at[pl.ds(so, EW_CHUNK)],
            w_hbm.at[pl.ds(w_base + c * EW_CHUNK, EW_CHUNK)],
            sem_so)

        @plsc.parallel_loop(0, EW_CHUNK, 16, unroll=8)
        def _sc_nxt(v):
            rbuf[pl.ds(so + v, 16)] = rbuf[pl.ds(so + v, 16)] * c_next

        to_d[c] = pltpu.async_copy(
            rbuf.at[pl.ds(so, EW_CHUNK)],
            t_sh.at[pl.ds(tile_base + c * EW_CHUNK, EW_CHUNK)],
            sem_o)
    for c in range(EW_CHUNKS - 2, EW_CHUNKS):
        wo_d[c].wait()
        to_d[c].wait()

    plsc.subcore_barrier()

    idx2_2d = idx2_hbm

    def one_round(i, carry):
        m = (_HSTEPS - 1) - i
        mm = lax.max(m - 1, 0)
        coef = tbl[mm, pl.ds(0, 16)]

        def _issue_in(b):
            return (
                pltpu.async_copy(
                    idx2_2d.at[pl.ds(i2_row0 + b * SC_BLK_ROWS, SC_BLK_ROWS)],
                    idx_ring.at[b % NRING], sem_in),
                pltpu.async_copy(
                    w_hbm.at[pl.ds(w_base + b * SC_BLK, SC_BLK)],
                    wval.at[b % NRING], sem_w),
            )

        def _drain_sc(p):
            pltpu.make_async_copy(
                rflat_hbm.at[pl.ds(0, SC_BLK)], wval.at[p], sem_sc).wait()

        ins = [_issue_in(b) for b in range(NRING - 1)]
        for b in range(SC_BLKS):
            nxt = b + NRING - 1
            if nxt < SC_BLKS:
                if nxt - NRING >= 0:
                    _drain_sc(nxt % NRING)
                ins.append(_issue_in(nxt))
            ins[b][0].wait()
            ins[b][1].wait()
            p = b % NRING
            for j in range(SC_BLK_ROWS):
                pltpu.async_copy(
                    wval.at[p, pl.ds(j * LANES, LANES)],
                    t_sh.at[idx_ring.at[p, j]],
                    sem_sc, add=True)
        for b in range(max(0, SC_BLKS - NRING), SC_BLKS):
            _drain_sc(b % NRING)

        plsc.subcore_barrier()

        rin2 = [pltpu.async_copy(
            rflat_hbm.at[pl.ds(tile_base, EW_CHUNK)],
            rbuf.at[pl.ds(0, EW_CHUNK)], sem_r)]
        st_in = [pltpu.async_copy(
            t_sh.at[pl.ds(tile_base, EW_CHUNK)],
            stage.at[pl.ds(0, EW_CHUNK)], sem_in)]
        wo2 = [None] * EW_CHUNKS
        to2 = [None] * EW_CHUNKS
        for c in range(EW_CHUNKS):
            so = (c % 3) * EW_CHUNK
            if c + 1 < EW_CHUNKS:
                if c - 1 >= 0:
                    wo2[c - 1].wait()
                    to2[c - 1].wait()
                no = ((c + 1) % 3) * EW_CHUNK
                rin2.append(pltpu.async_copy(
                    rflat_hbm.at[pl.ds(tile_base + (c + 1) * EW_CHUNK,
                                       EW_CHUNK)],
                    rbuf.at[pl.ds(no, EW_CHUNK)], sem_r))
                st_in.append(pltpu.async_copy(
                    t_sh.at[pl.ds(tile_base + (c + 1) * EW_CHUNK, EW_CHUNK)],
                    stage.at[pl.ds(no, EW_CHUNK)], sem_in))
            st_in[c].wait()
            wo2[c] = pltpu.async_copy(
                stage.at[pl.ds(so, EW_CHUNK)],
                w_hbm.at[pl.ds(w_base + c * EW_CHUNK, EW_CHUNK)], sem_so)
            rin2[c].wait()

            @plsc.parallel_loop(0, EW_CHUNK, 16, unroll=8)
            def _scale(v):
                rbuf[pl.ds(so + v, 16)] = rbuf[pl.ds(so + v, 16)] * coef

            to2[c] = pltpu.async_copy(
                rbuf.at[pl.ds(so, EW_CHUNK)],
                t_sh.at[pl.ds(tile_base + c * EW_CHUNK, EW_CHUNK)], sem_o)
        for c in range(EW_CHUNKS - 2, EW_CHUNKS):
            wo2[c].wait()
            to2[c].wait()

        plsc.subcore_barrier()
        return carry

    lax.fori_loop(0, _HSTEPS, one_round, 0)


@functools.partial(
    pl.kernel,
    out_type=jax.ShapeDtypeStruct((N_CELLS,), jnp.float32),
    mesh=mesh,
    scratch_types=[
        pltpu.VMEM_SHARED((N_CELLS,), jnp.float32),
        pltpu.VMEM((NRING, SC_BLK_ROWS, LANES), jnp.int32),
        pltpu.VMEM((NRING, SC_BLK), jnp.float32),
        pltpu.VMEM((3, EW_CHUNK), jnp.float32),
        pltpu.SemaphoreType.DMA,
        pltpu.SemaphoreType.DMA,
        pltpu.SemaphoreType.DMA,
        pltpu.SemaphoreType.DMA,
    ],
)
def _combine(w_hbm, idx2d_hbm, acc_out,
             acc_sh, idx_ring, uval, stage, sem_in, sem_w, sem_sc, sem_o):
    cid = lax.axis_index("c")
    sid = lax.axis_index("s")
    tile_row0 = sid * ROWS_PER_TILE
    tile_base = sid * PER_TILE

    @pl.when(cid == 0)
    def _init():
        pltpu.sync_copy(w_hbm.at[pl.ds(tile_base, PER_TILE)],
                        acc_sh.at[pl.ds(tile_base, PER_TILE)])

    plsc.subcore_barrier()

    @pl.when(cid == 0)
    def _scatter():
        def _issue_in(b):
            return (
                pltpu.async_copy(
                    idx2d_hbm.at[pl.ds(tile_row0 + b * SC_BLK_ROWS,
                                       SC_BLK_ROWS)],
                    idx_ring.at[b % NRING], sem_in),
                pltpu.async_copy(
                    w_hbm.at[pl.ds(N_CELLS + tile_base + b * SC_BLK, SC_BLK)],
                    uval.at[b % NRING], sem_w),
            )

        def _drain_sc(p):
            pltpu.make_async_copy(
                w_hbm.at[pl.ds(0, SC_BLK)], uval.at[p], sem_sc).wait()

        ins = [_issue_in(b) for b in range(NRING - 1)]
        for b in range(SC_BLKS):
            nxt = b + NRING - 1
            if nxt < SC_BLKS:
                if nxt - NRING >= 0:
                    _drain_sc(nxt % NRING)
                ins.append(_issue_in(nxt))
            ins[b][0].wait()
            ins[b][1].wait()
            p = b % NRING
            for j in range(SC_BLK_ROWS):
                pltpu.async_copy(
                    uval.at[p, pl.ds(j * LANES, LANES)],
                    acc_sh.at[idx_ring.at[p, j]],
                    sem_sc, add=True)
        for b in range(max(0, SC_BLKS - NRING), SC_BLKS):
            _drain_sc(b % NRING)

    plsc.subcore_barrier()

    @pl.when(cid == 0)
    def _writeout():
        pltpu.sync_copy(acc_sh.at[pl.ds(tile_base, PER_TILE)],
                        acc_out.at[pl.ds(tile_base, PER_TILE)])


@functools.partial(
    pl.kernel,
    out_type=[
        jax.ShapeDtypeStruct((N_CELLS,), jnp.float32),
        jax.ShapeDtypeStruct((N_CELLS,), jnp.float32),
    ],
    mesh=mesh,
    scratch_types=[
        pltpu.VMEM_SHARED((N_CELLS,), jnp.float32),
        pltpu.VMEM((NRING, SC_BLK_ROWS, LANES), jnp.int32),
        pltpu.VMEM((NRING, SC_BLK), jnp.float32),
        pltpu.VMEM((NRING, EW_CHUNK), jnp.float32),
        pltpu.VMEM((NRING, EW_CHUNK), jnp.float32),
        pltpu.VMEM((NRING, EW_CHUNK), jnp.float32),
        pltpu.VMEM((16,), jnp.int32),
        pltpu.SemaphoreType.DMA,
        pltpu.SemaphoreType.DMA,
        pltpu.SemaphoreType.DMA,
        pltpu.SemaphoreType.DMA,
        pltpu.SemaphoreType.DMA,
        pltpu.SemaphoreType.DMA,
    ],
)
def _route(rflat_hbm, idx2d_hbm, it_hbm, acc_out, curw,
           acc_sh, idx_buf, cur_buf, acc_stage, ew_cur, ew_out, it_v,
           sem_idx, sem_cin, sem_sc, sem_a, sem_c, sem_o):
    cid = lax.axis_index("c")
    sid = lax.axis_index("s")
    tile_row0 = sid * ROWS_PER_TILE
    tile_base = sid * PER_TILE

    pltpu.sync_copy(it_hbm, it_v)

    @pl.when(cid == 0)
    def _init():
        pltpu.sync_copy(rflat_hbm.at[pl.ds(tile_base, PER_TILE)],
                        acc_sh.at[pl.ds(tile_base, PER_TILE)])
        pltpu.sync_copy(rflat_hbm.at[pl.ds(tile_base, PER_TILE)],
                        curw.at[pl.ds(tile_base, PER_TILE)])

    plsc.subcore_barrier()

    def _issue_sc_in(b):
        return (
            pltpu.async_copy(
                idx2d_hbm.at[pl.ds(tile_row0 + b * SC_BLK_ROWS, SC_BLK_ROWS)],
                idx_buf.at[b % NRING], sem_idx),
            pltpu.async_copy(
                curw.at[pl.ds(tile_base + b * SC_BLK, SC_BLK)],
                cur_buf.at[b % NRING], sem_cin),
        )

    def _issue_ew_in(c):
        off = tile_base + c * EW_CHUNK
        return (
            pltpu.async_copy(acc_sh.at[pl.ds(off, EW_CHUNK)],
                             acc_stage.at[c % NRING], sem_a),
            pltpu.async_copy(curw.at[pl.ds(off, EW_CHUNK)],
                             ew_cur.at[c % NRING], sem_c),
        )

    def one_round(_, carry):
        @pl.when(cid == 0)
        def _scatter():
            def _drain_sc(p):
                pltpu.make_async_copy(
                    rflat_hbm.at[pl.ds(0, SC_BLK)],
                    cur_buf.at[p], sem_sc).wait()

            ins = [_issue_sc_in(b) for b in range(NRING - 1)]
            for b in range(SC_BLKS):
                nxt = b + NRING - 1
                if nxt < SC_BLKS:
                    if nxt - NRING >= 0:
                        _drain_sc(nxt % NRING)
                    ins.append(_issue_sc_in(nxt))
                ins[b][0].wait()
                ins[b][1].wait()
                p = b % NRING
                for j in range(SC_BLK_ROWS):
                    pltpu.async_copy(
                        cur_buf.at[p, pl.ds(j * LANES, LANES)],
                        acc_sh.at[idx_buf.at[p, j]],
                        sem_sc, add=True)
            for b in range(max(0, SC_BLKS - NRING), SC_BLKS):
                _drain_sc(b % NRING)

        plsc.subcore_barrier()

        @pl.when(cid == 0)
        def _elementwise():
            ins = [_issue_ew_in(c) for c in range(NRING - 1)]
            o_d = [None] * EW_CHUNKS
            for c in range(EW_CHUNKS):
                nxt = c + NRING - 1
                if nxt < EW_CHUNKS:
                    ins.append(_issue_ew_in(nxt))
                ins[c][0].wait()
                ins[c][1].wait()
                if c - NRING >= 0:
                    o_d[c - NRING].wait()
                p = c % NRING

                @plsc.parallel_loop(0, EW_CHUNK, 16, unroll=8)
                def _ew(v):
                    sl = pl.ds(v, 16)
                    ew_out[p, sl] = acc_stage[p, sl] - ew_cur[p, sl]

                o_d[c] = pltpu.async_copy(
                    ew_out.at[p],
                    curw.at[pl.ds(tile_base + c * EW_CHUNK, EW_CHUNK)],
                    sem_o)
            for c in range(max(0, EW_CHUNKS - NRING), EW_CHUNKS):
                o_d[c].wait()

        plsc.subcore_barrier()
        return carry

    n_rounds = it_v[pl.ds(0, 16)][0]
    lax.fori_loop(0, n_rounds, one_round, 0)

    @pl.when(cid == 0)
    def _writeout():
        pltpu.sync_copy(acc_sh.at[pl.ds(tile_base, PER_TILE)],
                        acc_out.at[pl.ds(tile_base, PER_TILE)])


def _tbl_host():
    t = jnp.zeros((2, 32, LANES), jnp.float32)
    e = (jnp.array(_EVEN, jnp.float32)[:, None]
         * jnp.ones((1, LANES), jnp.float32))
    o = (jnp.array(_ODD, jnp.float32)[:, None]
         * jnp.ones((1, LANES), jnp.float32))
    t = t.at[0, : len(_EVEN)].set(e)
    t = t.at[1, : len(_ODD)].set(o)
    return t


def kernel(runoff_generated, flow_direction_indices, iterations):
    h, w = runoff_generated.shape
    r_flat = runoff_generated.reshape(-1)
    idx_flat = flow_direction_indices.reshape(-1)
    idx_2d = flow_direction_indices.reshape(N_ROWS, LANES)

    def fast(_):
        tbl = _tbl_host()
        w_buf, _i2 = _horner(r_flat, idx_flat, tbl)
        return _combine(w_buf, idx_2d)

    def general(_):
        it = jnp.full((16,), iterations, dtype=jnp.int32)
        out, _cw = _route(r_flat, idx_2d, it)
        return out

    it_scalar = jnp.asarray(iterations, jnp.int32)
    out = lax.cond(it_scalar == POLY_ITERS, fast, general, 0)
    return out.reshape(h, w)


# speedup vs baseline: 2.4069x; 1.0886x over previous
import functools

import jax
import jax.numpy as jnp
from jax import lax
from jax.experimental import pallas as pl
from jax.experimental.pallas import tpu as pltpu
from jax.experimental.pallas import tpu_sc as plsc

LANES = 128
N_CELLS = 1024 * 1024
N_ROWS = N_CELLS // LANES
NUM_TILES = 16
PER_TILE = N_CELLS // NUM_TILES
ROWS_PER_TILE = PER_TILE // LANES

NRING = 4
SC_BLK_ROWS = 32
SC_BLKS = ROWS_PER_TILE // SC_BLK_ROWS
SC_BLK = SC_BLK_ROWS * LANES
EW_CHUNK = 2048
EW_CHUNKS = PER_TILE // EW_CHUNK

POLY_ITERS = 32


def _poly_coeffs(n):
    a, c = [1], [1]
    for _ in range(n):
        an = [0] * (len(c) + 1)
        cn = [0] * (len(c) + 1)
        for i, v in enumerate(a):
            an[i] += v
            cn[i] += v
        for i, v in enumerate(c):
            an[i + 1] += v
            cn[i + 1] += v
            cn[i] -= v
        a, c = an, cn
    return a


_A32 = _poly_coeffs(POLY_ITERS)
_EVEN = _A32[0::2]
_ODD = _A32[1::2] + [0]
_HSTEPS = len(_EVEN) - 1


mesh = plsc.VectorSubcoreMesh(
    core_axis_name="c", subcore_axis_name="s", num_cores=2, num_subcores=16
)


@functools.partial(
    pl.kernel,
    out_type=jax.ShapeDtypeStruct((N_ROWS, LANES), jnp.int32),
    mesh=mesh,
    scratch_types=[
        pltpu.VMEM_SHARED((N_CELLS,), jnp.int32),
        pltpu.VMEM((2 * SC_BLK,), jnp.int32),
        pltpu.VMEM((2, SC_BLK_ROWS, LANES), jnp.int32),
        pltpu.SemaphoreType.DMA,
        pltpu.SemaphoreType.DMA,
    ],
)
def _compose(idxflat_hbm, idx2_hbm, idx_sh, ibuf, gdst, sem_sc, sem_o):
    cid = lax.axis_index("c")
    sid = lax.axis_index("s")
    seg = N_CELLS // NUM_TILES
    pltpu.sync_copy(idxflat_hbm.at[pl.ds(sid * seg, seg)],
                    idx_sh.at[pl.ds(sid * seg, seg)])
    plsc.subcore_barrier()

    rows_p = N_ROWS // 2 // NUM_TILES
    row0 = cid * (N_ROWS // 2) + sid * rows_p
    nblk = rows_p // SC_BLK_ROWS
    o_d = [None] * nblk
    for b in range(nblk):
        p = b % 2
        pltpu.sync_copy(
            idx_sh.at[pl.ds((row0 + b * SC_BLK_ROWS) * LANES, SC_BLK)],
            ibuf.at[pl.ds(p * SC_BLK, SC_BLK)])
        if b - 2 >= 0:
            o_d[b - 2].wait()
        g = [pltpu.async_copy(
            idx_sh.at[ibuf.at[pl.ds(p * SC_BLK + j * LANES, LANES)]],
            gdst.at[p, j], sem_sc) for j in range(SC_BLK_ROWS)]
        for d in g:
            d.wait()
        o_d[b] = pltpu.async_copy(
            gdst.at[p],
            idx2_hbm.at[pl.ds(row0 + b * SC_BLK_ROWS, SC_BLK_ROWS)],
            sem_o)
    for b in range(max(0, nblk - 2), nblk):
        o_d[b].wait()


@functools.partial(
    pl.kernel,
    out_type=jax.ShapeDtypeStruct((2 * N_CELLS,), jnp.float32),
    mesh=mesh,
    scratch_types=[
        pltpu.VMEM_SHARED((N_CELLS,), jnp.float32),
        pltpu.VMEM((NRING, SC_BLK_ROWS, LANES), jnp.int32),
        pltpu.VMEM((NRING, SC_BLK), jnp.float32),
        pltpu.VMEM((3 * EW_CHUNK,), jnp.float32),
        pltpu.VMEM((3 * EW_CHUNK,), jnp.float32),
        pltpu.VMEM((32, LANES), jnp.float32),
        pltpu.SemaphoreType.DMA,
        pltpu.SemaphoreType.DMA,
        pltpu.SemaphoreType.DMA,
        pltpu.SemaphoreType.DMA,
        pltpu.SemaphoreType.DMA,
        pltpu.SemaphoreType.DMA,
    ],
)
def _horner(rflat_hbm, idx2_in_hbm, tbl_hbm, w_hbm,
            t_sh, idx_ring, wval, stage, rbuf, tbl,
            sem_in, sem_w, sem_sc, sem_so, sem_r, sem_o):
    cid = lax.axis_index("c")
    sid = lax.axis_index("s")
    tile_row0 = sid * ROWS_PER_TILE
    tile_base = sid * PER_TILE
    w_base = cid * N_CELLS + tile_base
    i2_row0 = tile_row0

    pltpu.sync_copy(tbl_hbm.at[cid], tbl)

    c_top = tbl[16, pl.ds(0, 16)]
    c_next = tbl[15, pl.ds(0, 16)]

    rin = [pltpu.async_copy(
        rflat_hbm.at[pl.ds(tile_base, EW_CHUNK)],
        rbuf.at[pl.ds(0, EW_CHUNK)], sem_r)]
    wo_d = [None] * EW_CHUNKS
    to_d = [None] * EW_CHUNKS
    for c in range(EW_CHUNKS):
        if c + 1 < EW_CHUNKS:
            if c - 1 >= 0:
                wo_d[c - 1].wait()
                to_d[c - 1].wait()
            rin.append(pltpu.async_copy(
                rflat_hbm.at[pl.ds(tile_base + (c + 1) * EW_CHUNK, EW_CHUNK)],
                rbuf.at[pl.ds(((c + 1) % 3) * EW_CHUNK, EW_CHUNK)], sem_r))
        rin[c].wait()
        so = (c % 3) * EW_CHUNK

        @plsc.parallel_loop(0, EW_CHUNK, 16, unroll=8)
        def _sc_top(v):
            stage[pl.ds(so + v, 16)] = rbuf[pl.ds(so + v, 16)] * c_top

        wo_d[c] = pltpu.async_copy(
            stage.at[pl.ds(so, EW_CHUNK)],
            w_hbm.at[pl.ds(w_base + c * EW_CHUNK, EW_CHUNK)],
            sem_so)

        @plsc.parallel_loop(0, EW_CHUNK, 16, unroll=8)
        def _sc_nxt(v):
            rbuf[pl.ds(so + v, 16)] = rbuf[pl.ds(so + v, 16)] * c_next

        to_d[c] = pltpu.async_copy(
            rbuf.at[pl.ds(so, EW_CHUNK)],
            t_sh.at[pl.ds(tile_base + c * EW_CHUNK, EW_CHUNK)],
            sem_o)
    for c in range(EW_CHUNKS - 2, EW_CHUNKS):
        wo_d[c].wait()
        to_d[c].wait()

    plsc.subcore_barrier()

    idx2_2d = idx2_in_hbm

    def one_round(i, carry):
        m = (_HSTEPS - 1) - i
        mm = lax.max(m - 1, 0)
        coef = tbl[mm, pl.ds(0, 16)]

        def _issue_in(b):
            return (
                pltpu.async_copy(
                    idx2_2d.at[pl.ds(i2_row0 + b * SC_BLK_ROWS, SC_BLK_ROWS)],
                    idx_ring.at[b % NRING], sem_in),
                pltpu.async_copy(
                    w_hbm.at[pl.ds(w_base + b * SC_BLK, SC_BLK)],
                    wval.at[b % NRING], sem_w),
            )

        def _drain_sc(p):
            pltpu.make_async_copy(
                rflat_hbm.at[pl.ds(0, SC_BLK)], wval.at[p], sem_sc).wait()

        ins = [_issue_in(b) for b in range(NRING - 1)]
        for b in range(SC_BLKS):
            nxt = b + NRING - 1
            if nxt < SC_BLKS:
                if nxt - NRING >= 0:
                    _drain_sc(nxt % NRING)
                ins.append(_issue_in(nxt))
            ins[b][0].wait()
            ins[b][1].wait()
            p = b % NRING
            for j in range(SC_BLK_ROWS):
                pltpu.async_copy(
                    wval.at[p, pl.ds(j * LANES, LANES)],
                    t_sh.at[idx_ring.at[p, j]],
                    sem_sc, add=True)
        for b in range(max(0, SC_BLKS - NRING), SC_BLKS):
            _drain_sc(b % NRING)

        plsc.subcore_barrier()

        rin2 = [pltpu.async_copy(
            rflat_hbm.at[pl.ds(tile_base, EW_CHUNK)],
            rbuf.at[pl.ds(0, EW_CHUNK)], sem_r)]
        st_in = [pltpu.async_copy(
            t_sh.at[pl.ds(tile_base, EW_CHUNK)],
            stage.at[pl.ds(0, EW_CHUNK)], sem_in)]
        wo2 = [None] * EW_CHUNKS
        to2 = [None] * EW_CHUNKS
        for c in range(EW_CHUNKS):
            so = (c % 3) * EW_CHUNK
            if c + 1 < EW_CHUNKS:
                if c - 1 >= 0:
                    wo2[c - 1].wait()
                    to2[c - 1].wait()
                no = ((c + 1) % 3) * EW_CHUNK
                rin2.append(pltpu.async_copy(
                    rflat_hbm.at[pl.ds(tile_base + (c + 1) * EW_CHUNK,
                                       EW_CHUNK)],
                    rbuf.at[pl.ds(no, EW_CHUNK)], sem_r))
                st_in.append(pltpu.async_copy(
                    t_sh.at[pl.ds(tile_base + (c + 1) * EW_CHUNK, EW_CHUNK)],
                    stage.at[pl.ds(no, EW_CHUNK)], sem_in))
            st_in[c].wait()
            wo2[c] = pltpu.async_copy(
                stage.at[pl.ds(so, EW_CHUNK)],
                w_hbm.at[pl.ds(w_base + c * EW_CHUNK, EW_CHUNK)], sem_so)
            rin2[c].wait()

            @plsc.parallel_loop(0, EW_CHUNK, 16, unroll=8)
            def _scale(v):
                rbuf[pl.ds(so + v, 16)] = rbuf[pl.ds(so + v, 16)] * coef

            to2[c] = pltpu.async_copy(
                rbuf.at[pl.ds(so, EW_CHUNK)],
                t_sh.at[pl.ds(tile_base + c * EW_CHUNK, EW_CHUNK)], sem_o)
        for c in range(EW_CHUNKS - 2, EW_CHUNKS):
            wo2[c].wait()
            to2[c].wait()

        plsc.subcore_barrier()
        return carry

    lax.fori_loop(0, _HSTEPS, one_round, 0)


@functools.partial(
    pl.kernel,
    out_type=jax.ShapeDtypeStruct((N_CELLS,), jnp.float32),
    mesh=mesh,
    scratch_types=[
        pltpu.VMEM_SHARED((N_CELLS,), jnp.float32),
        pltpu.VMEM((NRING, SC_BLK_ROWS, LANES), jnp.int32),
        pltpu.VMEM((NRING, SC_BLK), jnp.float32),
        pltpu.VMEM((3, EW_CHUNK), jnp.float32),
        pltpu.SemaphoreType.DMA,
        pltpu.SemaphoreType.DMA,
        pltpu.SemaphoreType.DMA,
        pltpu.SemaphoreType.DMA,
    ],
)
def _combine(w_hbm, idx2d_hbm, acc_out,
             acc_sh, idx_ring, uval, stage, sem_in, sem_w, sem_sc, sem_o):
    cid = lax.axis_index("c")
    sid = lax.axis_index("s")
    tile_row0 = sid * ROWS_PER_TILE
    tile_base = sid * PER_TILE

    @pl.when(cid == 0)
    def _init():
        pltpu.sync_copy(w_hbm.at[pl.ds(tile_base, PER_TILE)],
                        acc_sh.at[pl.ds(tile_base, PER_TILE)])

    plsc.subcore_barrier()

    @pl.when(cid == 0)
    def _scatter():
        def _issue_in(b):
            return (
                pltpu.async_copy(
                    idx2d_hbm.at[pl.ds(tile_row0 + b * SC_BLK_ROWS,
                                       SC_BLK_ROWS)],
                    idx_ring.at[b % NRING], sem_in),
                pltpu.async_copy(
                    w_hbm.at[pl.ds(N_CELLS + tile_base + b * SC_BLK, SC_BLK)],
                    uval.at[b % NRING], sem_w),
            )

        def _drain_sc(p):
            pltpu.make_async_copy(
                w_hbm.at[pl.ds(0, SC_BLK)], uval.at[p], sem_sc).wait()

        ins = [_issue_in(b) for b in range(NRING - 1)]
        for b in range(SC_BLKS):
            nxt = b + NRING - 1
            if nxt < SC_BLKS:
                if nxt - NRING >= 0:
                    _drain_sc(nxt % NRING)
                ins.append(_issue_in(nxt))
            ins[b][0].wait()
            ins[b][1].wait()
            p = b % NRING
            for j in range(SC_BLK_ROWS):
                pltpu.async_copy(
                    uval.at[p, pl.ds(j * LANES, LANES)],
                    acc_sh.at[idx_ring.at[p, j]],
                    sem_sc, add=True)
        for b in range(max(0, SC_BLKS - NRING), SC_BLKS):
            _drain_sc(b % NRING)

    plsc.subcore_barrier()

    @pl.when(cid == 0)
    def _writeout():
        pltpu.sync_copy(acc_sh.at[pl.ds(tile_base, PER_TILE)],
                        acc_out.at[pl.ds(tile_base, PER_TILE)])


@functools.partial(
    pl.kernel,
    out_type=[
        jax.ShapeDtypeStruct((N_CELLS,), jnp.float32),
        jax.ShapeDtypeStruct((N_CELLS,), jnp.float32),
    ],
    mesh=mesh,
    scratch_types=[
        pltpu.VMEM_SHARED((N_CELLS,), jnp.float32),
        pltpu.VMEM((NRING, SC_BLK_ROWS, LANES), jnp.int32),
        pltpu.VMEM((NRING, SC_BLK), jnp.float32),
        pltpu.VMEM((NRING, EW_CHUNK), jnp.float32),
        pltpu.VMEM((NRING, EW_CHUNK), jnp.float32),
        pltpu.VMEM((NRING, EW_CHUNK), jnp.float32),
        pltpu.VMEM((16,), jnp.int32),
        pltpu.SemaphoreType.DMA,
        pltpu.SemaphoreType.DMA,
        pltpu.SemaphoreType.DMA,
        pltpu.SemaphoreType.DMA,
        pltpu.SemaphoreType.DMA,
        pltpu.SemaphoreType.DMA,
    ],
)
def _route(rflat_hbm, idx2d_hbm, it_hbm, acc_out, curw,
           acc_sh, idx_buf, cur_buf, acc_stage, ew_cur, ew_out, it_v,
           sem_idx, sem_cin, sem_sc, sem_a, sem_c, sem_o):
    cid = lax.axis_index("c")
    sid = lax.axis_index("s")
    tile_row0 = sid * ROWS_PER_TILE
    tile_base = sid * PER_TILE

    pltpu.sync_copy(it_hbm, it_v)

    @pl.when(cid == 0)
    def _init():
        pltpu.sync_copy(rflat_hbm.at[pl.ds(tile_base, PER_TILE)],
                        acc_sh.at[pl.ds(tile_base, PER_TILE)])
        pltpu.sync_copy(rflat_hbm.at[pl.ds(tile_base, PER_TILE)],
                        curw.at[pl.ds(tile_base, PER_TILE)])

    plsc.subcore_barrier()

    def _issue_sc_in(b):
        return (
            pltpu.async_copy(
                idx2d_hbm.at[pl.ds(tile_row0 + b * SC_BLK_ROWS, SC_BLK_ROWS)],
                idx_buf.at[b % NRING], sem_idx),
            pltpu.async_copy(
                curw.at[pl.ds(tile_base + b * SC_BLK, SC_BLK)],
                cur_buf.at[b % NRING], sem_cin),
        )

    def _issue_ew_in(c):
        off = tile_base + c * EW_CHUNK
        return (
            pltpu.async_copy(acc_sh.at[pl.ds(off, EW_CHUNK)],
                             acc_stage.at[c % NRING], sem_a),
            pltpu.async_copy(curw.at[pl.ds(off, EW_CHUNK)],
                             ew_cur.at[c % NRING], sem_c),
        )

    def one_round(_, carry):
        @pl.when(cid == 0)
        def _scatter():
            def _drain_sc(p):
                pltpu.make_async_copy(
                    rflat_hbm.at[pl.ds(0, SC_BLK)],
                    cur_buf.at[p], sem_sc).wait()

            ins = [_issue_sc_in(b) for b in range(NRING - 1)]
            for b in range(SC_BLKS):
                nxt = b + NRING - 1
                if nxt < SC_BLKS:
                    if nxt - NRING >= 0:
                        _drain_sc(nxt % NRING)
                    ins.append(_issue_sc_in(nxt))
                ins[b][0].wait()
                ins[b][1].wait()
                p = b % NRING
                for j in range(SC_BLK_ROWS):
                    pltpu.async_copy(
                        cur_buf.at[p, pl.ds(j * LANES, LANES)],
                        acc_sh.at[idx_buf.at[p, j]],
                        sem_sc, add=True)
            for b in range(max(0, SC_BLKS - NRING), SC_BLKS):
                _drain_sc(b % NRING)

        plsc.subcore_barrier()

        @pl.when(cid == 0)
        def _elementwise():
            ins = [_issue_ew_in(c) for c in range(NRING - 1)]
            o_d = [None] * EW_CHUNKS
            for c in range(EW_CHUNKS):
                nxt = c + NRING - 1
                if nxt < EW_CHUNKS:
                    ins.append(_issue_ew_in(nxt))
                ins[c][0].wait()
                ins[c][1].wait()
                if c - NRING >= 0:
                    o_d[c - NRING].wait()
                p = c % NRING

                @plsc.parallel_loop(0, EW_CHUNK, 16, unroll=8)
                def _ew(v):
                    sl = pl.ds(v, 16)
                    ew_out[p, sl] = acc_stage[p, sl] - ew_cur[p, sl]

                o_d[c] = pltpu.async_copy(
                    ew_out.at[p],
                    curw.at[pl.ds(tile_base + c * EW_CHUNK, EW_CHUNK)],
                    sem_o)
            for c in range(max(0, EW_CHUNKS - NRING), EW_CHUNKS):
                o_d[c].wait()

        plsc.subcore_barrier()
        return carry

    n_rounds = it_v[pl.ds(0, 16)][0]
    lax.fori_loop(0, n_rounds, one_round, 0)

    @pl.when(cid == 0)
    def _writeout():
        pltpu.sync_copy(acc_sh.at[pl.ds(tile_base, PER_TILE)],
                        acc_out.at[pl.ds(tile_base, PER_TILE)])


def _tbl_host():
    t = jnp.zeros((2, 32, LANES), jnp.float32)
    e = (jnp.array(_EVEN, jnp.float32)[:, None]
         * jnp.ones((1, LANES), jnp.float32))
    o = (jnp.array(_ODD, jnp.float32)[:, None]
         * jnp.ones((1, LANES), jnp.float32))
    t = t.at[0, : len(_EVEN)].set(e)
    t = t.at[1, : len(_ODD)].set(o)
    return t


def kernel(runoff_generated, flow_direction_indices, iterations):
    h, w = runoff_generated.shape
    r_flat = runoff_generated.reshape(-1)
    idx_flat = flow_direction_indices.reshape(-1)
    idx_2d = flow_direction_indices.reshape(N_ROWS, LANES)

    def fast(_):
        tbl = _tbl_host()
        idx2 = _compose(idx_flat)
        w_buf = _horner(r_flat, idx2, tbl)
        return _combine(w_buf, idx_2d)

    def general(_):
        it = jnp.full((16,), iterations, dtype=jnp.int32)
        out, _cw = _route(r_flat, idx_2d, it)
        return out

    it_scalar = jnp.asarray(iterations, jnp.int32)
    out = lax.cond(it_scalar == POLY_ITERS, fast, general, 0)
    return out.reshape(h, w)


# speedup vs baseline: 2.7297x; 1.1341x over previous
import functools

import jax
import jax.numpy as jnp
from jax import lax
from jax.experimental import pallas as pl
from jax.experimental.pallas import tpu as pltpu
from jax.experimental.pallas import tpu_sc as plsc

LANES = 128
N_CELLS = 1024 * 1024
N_ROWS = N_CELLS // LANES
NUM_TILES = 16
PER_TILE = N_CELLS // NUM_TILES
ROWS_PER_TILE = PER_TILE // LANES

NRING = 4
SC_BLK_ROWS = 32
SC_BLKS = ROWS_PER_TILE // SC_BLK_ROWS
SC_BLK = SC_BLK_ROWS * LANES
EW_CHUNK = 2048
EW_CHUNKS = PER_TILE // EW_CHUNK
FCH = 4096
FCHS = PER_TILE // FCH

POLY_ITERS = 32


def _poly_coeffs(n):
    a, c = [1], [1]
    for _ in range(n):
        an = [0] * (len(c) + 1)
        cn = [0] * (len(c) + 1)
        for i, v in enumerate(a):
            an[i] += v
            cn[i] += v
        for i, v in enumerate(c):
            an[i + 1] += v
            cn[i + 1] += v
            cn[i] -= v
        a, c = an, cn
    return a


_A32 = _poly_coeffs(POLY_ITERS)
_EVEN = _A32[0::2]
_ODD = _A32[1::2] + [0]
_HSTEPS = len(_EVEN) - 1


mesh = plsc.VectorSubcoreMesh(
    core_axis_name="c", subcore_axis_name="s", num_cores=2, num_subcores=16
)


@functools.partial(
    pl.kernel,
    out_type=jax.ShapeDtypeStruct((N_ROWS, LANES), jnp.int32),
    mesh=mesh,
    scratch_types=[
        pltpu.VMEM_SHARED((N_CELLS,), jnp.int32),
        pltpu.VMEM((2 * SC_BLK,), jnp.int32),
        pltpu.VMEM((2, SC_BLK_ROWS, LANES), jnp.int32),
        pltpu.SemaphoreType.DMA,
        pltpu.SemaphoreType.DMA,
    ],
)
def _compose(idxflat_hbm, idx2_hbm, idx_sh, ibuf, gdst, sem_sc, sem_o):
    cid = lax.axis_index("c")
    sid = lax.axis_index("s")
    seg = N_CELLS // NUM_TILES
    pltpu.sync_copy(idxflat_hbm.at[pl.ds(sid * seg, seg)],
                    idx_sh.at[pl.ds(sid * seg, seg)])
    plsc.subcore_barrier()

    rows_p = N_ROWS // 2 // NUM_TILES
    row0 = cid * (N_ROWS // 2) + sid * rows_p
    nblk = rows_p // SC_BLK_ROWS
    o_d = [None] * nblk
    for b in range(nblk):
        p = b % 2
        pltpu.sync_copy(
            idx_sh.at[pl.ds((row0 + b * SC_BLK_ROWS) * LANES, SC_BLK)],
            ibuf.at[pl.ds(p * SC_BLK, SC_BLK)])
        if b - 2 >= 0:
            o_d[b - 2].wait()
        g = [pltpu.async_copy(
            idx_sh.at[ibuf.at[pl.ds(p * SC_BLK + j * LANES, LANES)]],
            gdst.at[p, j], sem_sc) for j in range(SC_BLK_ROWS)]
        for d in g:
            d.wait()
        o_d[b] = pltpu.async_copy(
            gdst.at[p],
            idx2_hbm.at[pl.ds(row0 + b * SC_BLK_ROWS, SC_BLK_ROWS)],
            sem_o)
    for b in range(max(0, nblk - 2), nblk):
        o_d[b].wait()


@functools.partial(
    pl.kernel,
    out_type=jax.ShapeDtypeStruct((2 * N_CELLS,), jnp.float32),
    mesh=mesh,
    scratch_types=[
        pltpu.VMEM_SHARED((N_CELLS,), jnp.float32),
        pltpu.VMEM((NRING, SC_BLK_ROWS, LANES), jnp.int32),
        pltpu.VMEM((NRING, SC_BLK), jnp.float32),
        pltpu.VMEM((3 * FCH,), jnp.float32),
        pltpu.VMEM((3 * FCH,), jnp.float32),
        pltpu.VMEM((32, LANES), jnp.float32),
        pltpu.SemaphoreType.DMA,
        pltpu.SemaphoreType.DMA,
        pltpu.SemaphoreType.DMA,
        pltpu.SemaphoreType.DMA,
        pltpu.SemaphoreType.DMA,
        pltpu.SemaphoreType.DMA,
    ],
)
def _horner(rflat_hbm, idx2_in_hbm, tbl_hbm, w_hbm,
            t_sh, idx_ring, wval, stage, rbuf, tbl,
            sem_in, sem_w, sem_sc, sem_so, sem_r, sem_o):
    cid = lax.axis_index("c")
    sid = lax.axis_index("s")
    tile_row0 = sid * ROWS_PER_TILE
    tile_base = sid * PER_TILE
    w_base = cid * N_CELLS + tile_base
    i2_row0 = tile_row0

    pltpu.sync_copy(tbl_hbm.at[cid], tbl)

    c_top = tbl[16, pl.ds(0, 16)]
    c_next = tbl[15, pl.ds(0, 16)]

    rin = [pltpu.async_copy(
        rflat_hbm.at[pl.ds(tile_base, FCH)],
        rbuf.at[pl.ds(0, FCH)], sem_r)]
    wo_d = [None] * FCHS
    to_d = [None] * FCHS
    for c in range(FCHS):
        if c + 1 < FCHS:
            if c - 1 >= 0:
                wo_d[c - 1].wait()
                to_d[c - 1].wait()
            rin.append(pltpu.async_copy(
                rflat_hbm.at[pl.ds(tile_base + (c + 1) * FCH, FCH)],
                rbuf.at[pl.ds(((c + 1) % 3) * FCH, FCH)], sem_r))
        rin[c].wait()
        so = (c % 3) * FCH

        @plsc.parallel_loop(0, FCH, 16, unroll=8)
        def _sc_top(v):
            stage[pl.ds(so + v, 16)] = rbuf[pl.ds(so + v, 16)] * c_top

        wo_d[c] = pltpu.async_copy(
            stage.at[pl.ds(so, FCH)],
            w_hbm.at[pl.ds(w_base + c * FCH, FCH)],
            sem_so)

        @plsc.parallel_loop(0, FCH, 16, unroll=8)
        def _sc_nxt(v):
            rbuf[pl.ds(so + v, 16)] = rbuf[pl.ds(so + v, 16)] * c_next

        to_d[c] = pltpu.async_copy(
            rbuf.at[pl.ds(so, FCH)],
            t_sh.at[pl.ds(tile_base + c * FCH, FCH)],
            sem_o)
    for c in range(FCHS - 2, FCHS):
        wo_d[c].wait()
        to_d[c].wait()

    plsc.subcore_barrier()

    idx2_2d = idx2_in_hbm

    def one_round(i, carry):
        m = (_HSTEPS - 1) - i
        mm = lax.max(m - 1, 0)
        coef = tbl[mm, pl.ds(0, 16)]

        def _issue_in(b):
            return (
                pltpu.async_copy(
                    idx2_2d.at[pl.ds(i2_row0 + b * SC_BLK_ROWS, SC_BLK_ROWS)],
                    idx_ring.at[b % NRING], sem_in),
                pltpu.async_copy(
                    w_hbm.at[pl.ds(w_base + b * SC_BLK, SC_BLK)],
                    wval.at[b % NRING], sem_w),
            )

        def _drain_sc(p):
            pltpu.make_async_copy(
                rflat_hbm.at[pl.ds(0, SC_BLK)], wval.at[p], sem_sc).wait()

        ins = [_issue_in(b) for b in range(NRING - 1)]
        for b in range(SC_BLKS):
            nxt = b + NRING - 1
            if nxt < SC_BLKS:
                if nxt - NRING >= 0:
                    _drain_sc(nxt % NRING)
                ins.append(_issue_in(nxt))
            ins[b][0].wait()
            ins[b][1].wait()
            p = b % NRING
            for j in range(SC_BLK_ROWS):
                pltpu.async_copy(
                    wval.at[p, pl.ds(j * LANES, LANES)],
                    t_sh.at[idx_ring.at[p, j]],
                    sem_sc, add=True)
        for b in range(max(0, SC_BLKS - NRING), SC_BLKS):
            _drain_sc(b % NRING)

        plsc.subcore_barrier()

        rin2 = [pltpu.async_copy(
            rflat_hbm.at[pl.ds(tile_base, FCH)],
            rbuf.at[pl.ds(0, FCH)], sem_r)]
        st_in = [pltpu.async_copy(
            t_sh.at[pl.ds(tile_base, FCH)],
            stage.at[pl.ds(0, FCH)], sem_in)]
        wo2 = [None] * FCHS
        to2 = [None] * FCHS
        for c in range(FCHS):
            so = (c % 3) * FCH
            if c + 1 < FCHS:
                if c - 1 >= 0:
                    wo2[c - 1].wait()
                    to2[c - 1].wait()
                no = ((c + 1) % 3) * FCH
                rin2.append(pltpu.async_copy(
                    rflat_hbm.at[pl.ds(tile_base + (c + 1) * FCH,
                                       FCH)],
                    rbuf.at[pl.ds(no, FCH)], sem_r))
                st_in.append(pltpu.async_copy(
                    t_sh.at[pl.ds(tile_base + (c + 1) * FCH, FCH)],
                    stage.at[pl.ds(no, FCH)], sem_in))
            st_in[c].wait()
            wo2[c] = pltpu.async_copy(
                stage.at[pl.ds(so, FCH)],
                w_hbm.at[pl.ds(w_base + c * FCH, FCH)], sem_so)
            rin2[c].wait()

            @plsc.parallel_loop(0, FCH, 16, unroll=8)
            def _scale(v):
                rbuf[pl.ds(so + v, 16)] = rbuf[pl.ds(so + v, 16)] * coef

            to2[c] = pltpu.async_copy(
                rbuf.at[pl.ds(so, FCH)],
                t_sh.at[pl.ds(tile_base + c * FCH, FCH)], sem_o)
        for c in range(FCHS - 2, FCHS):
            wo2[c].wait()
            to2[c].wait()

        plsc.subcore_barrier()
        return carry

    lax.fori_loop(0, _HSTEPS, one_round, 0)


@functools.partial(
    pl.kernel,
    out_type=jax.ShapeDtypeStruct((N_CELLS,), jnp.float32),
    mesh=mesh,
    scratch_types=[
        pltpu.VMEM_SHARED((N_CELLS,), jnp.float32),
        pltpu.VMEM((NRING, SC_BLK_ROWS, LANES), jnp.int32),
        pltpu.VMEM((NRING, SC_BLK), jnp.float32),
        pltpu.VMEM((3, EW_CHUNK), jnp.float32),
        pltpu.SemaphoreType.DMA,
        pltpu.SemaphoreType.DMA,
        pltpu.SemaphoreType.DMA,
        pltpu.SemaphoreType.DMA,
    ],
)
def _combine(w_hbm, idx2d_hbm, acc_out,
             acc_sh, idx_ring, uval, stage, sem_in, sem_w, sem_sc, sem_o):
    cid = lax.axis_index("c")
    sid = lax.axis_index("s")
    tile_row0 = sid * ROWS_PER_TILE
    tile_base = sid * PER_TILE

    @pl.when(cid == 0)
    def _init():
        pltpu.sync_copy(w_hbm.at[pl.ds(tile_base, PER_TILE)],
                        acc_sh.at[pl.ds(tile_base, PER_TILE)])

    plsc.subcore_barrier()

    @pl.when(cid == 0)
    def _scatter():
        def _issue_in(b):
            return (
                pltpu.async_copy(
                    idx2d_hbm.at[pl.ds(tile_row0 + b * SC_BLK_ROWS,
                                       SC_BLK_ROWS)],
                    idx_ring.at[b % NRING], sem_in),
                pltpu.async_copy(
                    w_hbm.at[pl.ds(N_CELLS + tile_base + b * SC_BLK, SC_BLK)],
                    uval.at[b % NRING], sem_w),
            )

        def _drain_sc(p):
            pltpu.make_async_copy(
                w_hbm.at[pl.ds(0, SC_BLK)], uval.at[p], sem_sc).wait()

        ins = [_issue_in(b) for b in range(NRING - 1)]
        for b in range(SC_BLKS):
            nxt = b + NRING - 1
            if nxt < SC_BLKS:
                if nxt - NRING >= 0:
                    _drain_sc(nxt % NRING)
                ins.append(_issue_in(nxt))
            ins[b][0].wait()
            ins[b][1].wait()
            p = b % NRING
            for j in range(SC_BLK_ROWS):
                pltpu.async_copy(
                    uval.at[p, pl.ds(j * LANES, LANES)],
                    acc_sh.at[idx_ring.at[p, j]],
                    sem_sc, add=True)
        for b in range(max(0, SC_BLKS - NRING), SC_BLKS):
            _drain_sc(b % NRING)

    plsc.subcore_barrier()

    @pl.when(cid == 0)
    def _writeout():
        pltpu.sync_copy(acc_sh.at[pl.ds(tile_base, PER_TILE)],
                        acc_out.at[pl.ds(tile_base, PER_TILE)])


@functools.partial(
    pl.kernel,
    out_type=[
        jax.ShapeDtypeStruct((N_CELLS,), jnp.float32),
        jax.ShapeDtypeStruct((N_CELLS,), jnp.float32),
    ],
    mesh=mesh,
    scratch_types=[
        pltpu.VMEM_SHARED((N_CELLS,), jnp.float32),
        pltpu.VMEM((NRING, SC_BLK_ROWS, LANES), jnp.int32),
        pltpu.VMEM((NRING, SC_BLK), jnp.float32),
        pltpu.VMEM((NRING, EW_CHUNK), jnp.float32),
        pltpu.VMEM((NRING, EW_CHUNK), jnp.float32),
        pltpu.VMEM((NRING, EW_CHUNK), jnp.float32),
        pltpu.VMEM((16,), jnp.int32),
        pltpu.SemaphoreType.DMA,
        pltpu.SemaphoreType.DMA,
        pltpu.SemaphoreType.DMA,
        pltpu.SemaphoreType.DMA,
        pltpu.SemaphoreType.DMA,
        pltpu.SemaphoreType.DMA,
    ],
)
def _route(rflat_hbm, idx2d_hbm, it_hbm, acc_out, curw,
           acc_sh, idx_buf, cur_buf, acc_stage, ew_cur, ew_out, it_v,
           sem_idx, sem_cin, sem_sc, sem_a, sem_c, sem_o):
    cid = lax.axis_index("c")
    sid = lax.axis_index("s")
    tile_row0 = sid * ROWS_PER_TILE
    tile_base = sid * PER_TILE

    pltpu.sync_copy(it_hbm, it_v)

    @pl.when(cid == 0)
    def _init():
        pltpu.sync_copy(rflat_hbm.at[pl.ds(tile_base, PER_TILE)],
                        acc_sh.at[pl.ds(tile_base, PER_TILE)])
        pltpu.sync_copy(rflat_hbm.at[pl.ds(tile_base, PER_TILE)],
                        curw.at[pl.ds(tile_base, PER_TILE)])

    plsc.subcore_barrier()

    def _issue_sc_in(b):
        return (
            pltpu.async_copy(
                idx2d_hbm.at[pl.ds(tile_row0 + b * SC_BLK_ROWS, SC_BLK_ROWS)],
                idx_buf.at[b % NRING], sem_idx),
            pltpu.async_copy(
                curw.at[pl.ds(tile_base + b * SC_BLK, SC_BLK)],
                cur_buf.at[b % NRING], sem_cin),
        )

    def _issue_ew_in(c):
        off = tile_base + c * EW_CHUNK
        return (
            pltpu.async_copy(acc_sh.at[pl.ds(off, EW_CHUNK)],
                             acc_stage.at[c % NRING], sem_a),
            pltpu.async_copy(curw.at[pl.ds(off, EW_CHUNK)],
                             ew_cur.at[c % NRING], sem_c),
        )

    def one_round(_, carry):
        @pl.when(cid == 0)
        def _scatter():
            def _drain_sc(p):
                pltpu.make_async_copy(
                    rflat_hbm.at[pl.ds(0, SC_BLK)],
                    cur_buf.at[p], sem_sc).wait()

            ins = [_issue_sc_in(b) for b in range(NRING - 1)]
            for b in range(SC_BLKS):
                nxt = b + NRING - 1
                if nxt < SC_BLKS:
                    if nxt - NRING >= 0:
                        _drain_sc(nxt % NRING)
                    ins.append(_issue_sc_in(nxt))
                ins[b][0].wait()
                ins[b][1].wait()
                p = b % NRING
                for j in range(SC_BLK_ROWS):
                    pltpu.async_copy(
                        cur_buf.at[p, pl.ds(j * LANES, LANES)],
                        acc_sh.at[idx_buf.at[p, j]],
                        sem_sc, add=True)
            for b in range(max(0, SC_BLKS - NRING), SC_BLKS):
                _drain_sc(b % NRING)

        plsc.subcore_barrier()

        @pl.when(cid == 0)
        def _elementwise():
            ins = [_issue_ew_in(c) for c in range(NRING - 1)]
            o_d = [None] * EW_CHUNKS
            for c in range(EW_CHUNKS):
                nxt = c + NRING - 1
                if nxt < EW_CHUNKS:
                    ins.append(_issue_ew_in(nxt))
                ins[c][0].wait()
                ins[c][1].wait()
                if c - NRING >= 0:
                    o_d[c - NRING].wait()
                p = c % NRING

                @plsc.parallel_loop(0, EW_CHUNK, 16, unroll=8)
                def _ew(v):
                    sl = pl.ds(v, 16)
                    ew_out[p, sl] = acc_stage[p, sl] - ew_cur[p, sl]

                o_d[c] = pltpu.async_copy(
                    ew_out.at[p],
                    curw.at[pl.ds(tile_base + c * EW_CHUNK, EW_CHUNK)],
                    sem_o)
            for c in range(max(0, EW_CHUNKS - NRING), EW_CHUNKS):
                o_d[c].wait()

        plsc.subcore_barrier()
        return carry

    n_rounds = it_v[pl.ds(0, 16)][0]
    lax.fori_loop(0, n_rounds, one_round, 0)

    @pl.when(cid == 0)
    def _writeout():
        pltpu.sync_copy(acc_sh.at[pl.ds(tile_base, PER_TILE)],
                        acc_out.at[pl.ds(tile_base, PER_TILE)])


def _tbl_host():
    t = jnp.zeros((2, 32, LANES), jnp.float32)
    e = (jnp.array(_EVEN, jnp.float32)[:, None]
         * jnp.ones((1, LANES), jnp.float32))
    o = (jnp.array(_ODD, jnp.float32)[:, None]
         * jnp.ones((1, LANES), jnp.float32))
    t = t.at[0, : len(_EVEN)].set(e)
    t = t.at[1, : len(_ODD)].set(o)
    return t


def kernel(runoff_generated, flow_direction_indices, iterations):
    h, w = runoff_generated.shape
    r_flat = runoff_generated.reshape(-1)
    idx_flat = flow_direction_indices.reshape(-1)
    idx_2d = flow_direction_indices.reshape(N_ROWS, LANES)

    def fast(_):
        tbl = _tbl_host()
        idx2 = _compose(idx_flat)
        w_buf = _horner(r_flat, idx2, tbl)
        return _combine(w_buf, idx_2d)

    def general(_):
        it = jnp.full((16,), iterations, dtype=jnp.int32)
        out, _cw = _route(r_flat, idx_2d, it)
        return out

    it_scalar = jnp.asarray(iterations, jnp.int32)
    out = lax.cond(it_scalar == POLY_ITERS, fast, general, 0)
    return out.reshape(h, w)
